# baseline (speedup 1.0000x reference)
import jax
import jax.numpy as jnp
from jax import lax
from jax.experimental import pallas as pl
from jax.experimental.pallas import tpu as pltpu
from jax.experimental.pallas import tpu_sc as plsc

N = 10000
E = 320000
H = 128

NC = 2
NS = 16
NW = NC * NS
EPW = E // NW
C = 40
CPW = EPW // C
CPB = 25
EPB = C * CPB
NBLK = CPW // CPB
NP = 10240
RPT = NP // NS
RCH = 40
NRCH = RPT // RCH


def _edge_body(k_hbm, qv_hbm, src2_hbm, dst2_hbm, zeros_hbm, out_hbm,
               sidx, didx, kd, qvd, gsems, ssems, aggsh):
    cid = lax.axis_index("c")
    sid = lax.axis_index("s")
    wid = sid * NC + cid

    row0 = sid * RPT
    for c in range(NRCH):
        pltpu.sync_copy(zeros_hbm, aggsh.at[pl.ds(row0 + c * RCH, RCH)])
    plsc.subcore_barrier()

    crow0 = wid * CPW

    def start_gathers(j, b):
        sl = pl.ds(j * C, C)
        pltpu.async_copy(k_hbm.at[didx.at[sl]], kd.at[b], gsems[b])
        pltpu.async_copy(qv_hbm.at[sidx.at[sl]], qvd.at[b], gsems[b])

    def wait_gathers(j, b):
        sl = pl.ds(j * C, C)
        pltpu.make_async_copy(k_hbm.at[didx.at[sl]], kd.at[b],
                              gsems[b]).wait()
        pltpu.make_async_copy(qv_hbm.at[sidx.at[sl]], qvd.at[b],
                              gsems[b]).wait()

    def start_scatter(j, b):
        pltpu.async_copy(kd.at[b], aggsh.at[didx.at[pl.ds(j * C, C)]],
                         ssems[b], add=True)

    def wait_scatter(j, b):
        pltpu.make_async_copy(kd.at[b], aggsh.at[didx.at[pl.ds(j * C, C)]],
                              ssems[b]).wait()

    def block_body(nb, carry):
        base = wid * EPW + nb * EPB
        pltpu.sync_copy(src2_hbm.at[pl.ds(base, EPB)], sidx)
        pltpu.sync_copy(dst2_hbm.at[pl.ds(base, EPB)], didx)
        for j in range(CPB):
            b = j % 2
            start_gathers(j, b)
            wait_gathers(j, b)

            def edge_one(e, c2):
                for jj in range(H // 16):
                    sl = pl.ds(jj * 16, 16)
                    kk = kd[b, e, sl]
                    qq = qvd[b, e, sl]
                    vv = qvd[b, e, pl.ds(H + jj * 16, 16)]
                    em = jnp.exp(-(kk + qq))
                    kd[b, e, sl] = vv / (1.0 + em)
                return c2

            lax.fori_loop(0, C, edge_one, 0, unroll=False)
            start_scatter(j, b)
            wait_scatter(j, b)
        return carry

    lax.fori_loop(0, NBLK, block_body, 0, unroll=False)
    plsc.subcore_barrier()

    zbuf = kd.at[0, pl.ds(0, RCH)]
    for c in range(NRCH):
        r = row0 + c * RCH
        pltpu.sync_copy(aggsh.at[pl.ds(r, RCH)], zbuf)
        pltpu.sync_copy(zbuf, out_hbm.at[cid, pl.ds(r, RCH)])


@jax.jit
def _edge_pass(k, qv, src2, dst2, zeros):
    mesh = plsc.VectorSubcoreMesh(core_axis_name="c", subcore_axis_name="s")
    f = pl.kernel(
        _edge_body,
        out_type=jax.ShapeDtypeStruct((NC, NP, H), jnp.float32),
        mesh=mesh,
        scratch_types=[
            pltpu.VMEM((EPB,), jnp.int32),
            pltpu.VMEM((EPB,), jnp.int32),
            pltpu.VMEM((2, C, H), jnp.float32),
            pltpu.VMEM((2, C, 2 * H), jnp.float32),
            [pltpu.SemaphoreType.DMA, pltpu.SemaphoreType.DMA],
            [pltpu.SemaphoreType.DMA, pltpu.SemaphoreType.DMA],
            pltpu.VMEM_SHARED((NP, H), jnp.float32),
        ],
    )
    return f(k, qv, src2, dst2, zeros)


BLK = 2000


def _dense0_body(x_ref, wp_ref, bp_ref, wc_ref, bc_ref,
                 k_ref, qv_ref, s_ref):
    h = jnp.maximum(jnp.dot(x_ref[...], wp_ref[...],
                            preferred_element_type=jnp.float32)
                    + bp_ref[...], 0.0)
    out = jnp.dot(h, wc_ref[...],
                  preferred_element_type=jnp.float32) + bc_ref[...]
    k_ref[...] = out[:, 0:H]
    qv_ref[...] = out[:, H:3 * H]
    s_ref[...] = out[:, 3 * H:4 * H]


@jax.jit
def _dense0(x, wp, bp, wc, bc):
    return pl.pallas_call(
        _dense0_body,
        grid=(N // BLK,),
        in_specs=[
            pl.BlockSpec((BLK, H), lambda i: (i, 0)),
            pl.BlockSpec((H, H), lambda i: (0, 0)),
            pl.BlockSpec((1, H), lambda i: (0, 0)),
            pl.BlockSpec((H, 4 * H), lambda i: (0, 0)),
            pl.BlockSpec((1, 4 * H), lambda i: (0, 0)),
        ],
        out_specs=[
            pl.BlockSpec((BLK, H), lambda i: (i, 0)),
            pl.BlockSpec((BLK, 2 * H), lambda i: (i, 0)),
            pl.BlockSpec((BLK, H), lambda i: (i, 0)),
        ],
        out_shape=[
            jax.ShapeDtypeStruct((N, H), jnp.float32),
            jax.ShapeDtypeStruct((N, 2 * H), jnp.float32),
            jax.ShapeDtypeStruct((N, H), jnp.float32),
        ],
    )(x, wp, bp, wc, bc)


def _stats_body(a0_ref, a1_ref, s_ref, pre_ref, sum_ref, sq_ref):
    i = pl.program_id(0)
    pre = a0_ref[...] + a1_ref[...] + s_ref[...]
    pre_ref[...] = pre
    bs = jnp.sum(pre, axis=0, keepdims=True)
    bq = jnp.sum(pre * pre, axis=0, keepdims=True)

    @pl.when(i == 0)
    def _():
        sum_ref[...] = bs
        sq_ref[...] = bq

    @pl.when(i > 0)
    def _():
        sum_ref[...] += bs
        sq_ref[...] += bq


@jax.jit
def _stats(a0, a1, s):
    return pl.pallas_call(
        _stats_body,
        grid=(N // BLK,),
        in_specs=[pl.BlockSpec((BLK, H), lambda i: (i, 0))] * 3,
        out_specs=[
            pl.BlockSpec((BLK, H), lambda i: (i, 0)),
            pl.BlockSpec((1, H), lambda i: (0, 0)),
            pl.BlockSpec((1, H), lambda i: (0, 0)),
        ],
        out_shape=[
            jax.ShapeDtypeStruct((N, H), jnp.float32),
            jax.ShapeDtypeStruct((1, H), jnp.float32),
            jax.ShapeDtypeStruct((1, H), jnp.float32),
        ],
    )(a0, a1, s)


def _normproj_body(pre_ref, sum_ref, sq_ref, g_ref, be_ref, wc_ref, bc_ref,
                   k_ref, qv_ref, s_ref):
    mu = sum_ref[...] / N
    var = sq_ref[...] / N - mu * mu
    scale = g_ref[...] * lax.rsqrt(var + 1e-5)
    h = jnp.maximum((pre_ref[...] - mu) * scale + be_ref[...], 0.0)
    out = jnp.dot(h, wc_ref[...],
                  preferred_element_type=jnp.float32) + bc_ref[...]
    k_ref[...] = out[:, 0:H]
    qv_ref[...] = out[:, H:3 * H]
    s_ref[...] = out[:, 3 * H:4 * H]


@jax.jit
def _normproj(pre, sm, sq, g, be, wc, bc):
    return pl.pallas_call(
        _normproj_body,
        grid=(N // BLK,),
        in_specs=[
            pl.BlockSpec((BLK, H), lambda i: (i, 0)),
            pl.BlockSpec((1, H), lambda i: (0, 0)),
            pl.BlockSpec((1, H), lambda i: (0, 0)),
            pl.BlockSpec((1, H), lambda i: (0, 0)),
            pl.BlockSpec((1, H), lambda i: (0, 0)),
            pl.BlockSpec((H, 4 * H), lambda i: (0, 0)),
            pl.BlockSpec((1, 4 * H), lambda i: (0, 0)),
        ],
        out_specs=[
            pl.BlockSpec((BLK, H), lambda i: (i, 0)),
            pl.BlockSpec((BLK, 2 * H), lambda i: (i, 0)),
            pl.BlockSpec((BLK, H), lambda i: (i, 0)),
        ],
        out_shape=[
            jax.ShapeDtypeStruct((N, H), jnp.float32),
            jax.ShapeDtypeStruct((N, 2 * H), jnp.float32),
            jax.ShapeDtypeStruct((N, H), jnp.float32),
        ],
    )(pre, sm, sq, g, be, wc, bc)


def _head_body(pre_ref, sum_ref, sq_ref, g_ref, be_ref, wc_ref, bc_ref,
               out_ref):
    mu = sum_ref[...] / N
    var = sq_ref[...] / N - mu * mu
    scale = g_ref[...] * lax.rsqrt(var + 1e-5)
    h = jnp.maximum((pre_ref[...] - mu) * scale + be_ref[...], 0.0)
    out_ref[...] = jnp.dot(h, wc_ref[...],
                           preferred_element_type=jnp.float32) + bc_ref[...]


@jax.jit
def _head(pre, sm, sq, g, be, wc, bc):
    m = wc.shape[1]
    return pl.pallas_call(
        _head_body,
        grid=(N // BLK,),
        in_specs=[
            pl.BlockSpec((BLK, H), lambda i: (i, 0)),
            pl.BlockSpec((1, H), lambda i: (0, 0)),
            pl.BlockSpec((1, H), lambda i: (0, 0)),
            pl.BlockSpec((1, H), lambda i: (0, 0)),
            pl.BlockSpec((1, H), lambda i: (0, 0)),
            pl.BlockSpec((H, m), lambda i: (0, 0)),
            pl.BlockSpec((1, m), lambda i: (0, 0)),
        ],
        out_specs=pl.BlockSpec((BLK, m), lambda i: (i, 0)),
        out_shape=jax.ShapeDtypeStruct((N, m), jnp.float32),
    )(pre, sm, sq, g, be, wc, bc)


def _wcat(c):
    wc = jnp.concatenate([c['Wk'], c['Wq'], c['Wv'], c['Ws']], axis=1)
    bc = jnp.concatenate([c['bk'], c['bq'], c['bv'], c['b']])[None, :]
    return wc, bc


def kernel(x, ei, params):
    p = params
    zeros = jnp.zeros((RCH, H), jnp.float32)
    src2 = ei[0]
    dst2 = ei[1]

    wc1, bc1 = _wcat(p['c1'])
    k, qv, s = _dense0(x, p['Wp'], p['bp'][None, :], wc1, bc1)

    for i in (1, 2, 3):
        aggp = _edge_pass(k, qv, src2, dst2, zeros)
        pre, sm, sq = _stats(aggp[0, :N], aggp[1, :N], s)
        g = p['g%d' % i][None, :]
        be = p['be%d' % i][None, :]
        if i < 3:
            wc, bc = _wcat(p['c%d' % (i + 1)])
            k, qv, s = _normproj(pre, sm, sq, g, be, wc, bc)
        else:
            out = _head(pre, sm, sq, g, be, p['Wh'], p['bh'][None, :])
    return out


# speedup vs baseline: 1.1808x; 1.1808x over previous
import jax
import jax.numpy as jnp
from jax import lax
from jax.experimental import pallas as pl
from jax.experimental.pallas import tpu as pltpu
from jax.experimental.pallas import tpu_sc as plsc

N = 10000
E = 320000
H = 128

NC = 2
NS = 16
NW = NC * NS
EPW = E // NW
C = 40
CPW = EPW // C
CPB = 50
EPB = C * CPB
NBLK = CPW // CPB
NP = 10240
RPT = NP // NS
RCH = 40
NRCH = RPT // RCH


def _edge_body(k_hbm, qv_hbm, src2_hbm, dst2_hbm, zeros_hbm, out_hbm,
               sidx, didx, kd, qvd, gsems, ssems, aggsh):
    cid = lax.axis_index("c")
    sid = lax.axis_index("s")
    wid = sid * NC + cid

    row0 = sid * RPT
    for c in range(NRCH):
        pltpu.sync_copy(zeros_hbm, aggsh.at[pl.ds(row0 + c * RCH, RCH)])
    plsc.subcore_barrier()

    crow0 = wid * CPW

    def start_gathers(j, b):
        sl = pl.ds(j * C, C)
        pltpu.async_copy(k_hbm.at[didx.at[sl]], kd.at[b], gsems[b])
        pltpu.async_copy(qv_hbm.at[sidx.at[sl]], qvd.at[b], gsems[b])

    def wait_gathers(j, b):
        sl = pl.ds(j * C, C)
        pltpu.make_async_copy(k_hbm.at[didx.at[sl]], kd.at[b],
                              gsems[b]).wait()
        pltpu.make_async_copy(qv_hbm.at[sidx.at[sl]], qvd.at[b],
                              gsems[b]).wait()

    def start_scatter(j, b):
        pltpu.async_copy(kd.at[b], aggsh.at[didx.at[pl.ds(j * C, C)]],
                         ssems[b], add=True)

    def wait_scatter(b):
        pltpu.make_async_copy(kd.at[b], aggsh.at[didx.at[pl.ds(0, C)]],
                              ssems[b]).wait()

    def compute_chunk(b):
        kb = kd.at[b]
        qb = qvd.at[b]

        def edge_one(e, c2):
            for jj in range(H // 16):
                sl = pl.ds(jj * 16, 16)
                kk = kb[e, sl]
                qq = qb[e, sl]
                vv = qb[e, pl.ds(H + jj * 16, 16)]
                em = jnp.exp(-(kk + qq))
                kb[e, sl] = vv / (1.0 + em)
            return c2

        lax.fori_loop(0, C, edge_one, 0, unroll=4)

    def chunk_work(j, b, prefetch, drain):
        wait_gathers(j, b)
        if drain:
            wait_scatter(1 - b)
        if prefetch:
            start_gathers(j + 1, 1 - b)
        compute_chunk(b)
        start_scatter(j, b)

    def block_body(nb, carry):
        wait_scatter(1)
        base = wid * EPW + nb * EPB
        pltpu.sync_copy(src2_hbm.at[pl.ds(base, EPB)], sidx)
        pltpu.sync_copy(dst2_hbm.at[pl.ds(base, EPB)], didx)
        start_gathers(0, 0)
        chunk_work(0, 0, True, False)
        chunk_work(1, 1, True, True)

        def pair_body(t, c2):
            chunk_work(2 * t, 0, True, True)
            chunk_work(2 * t + 1, 1, True, True)
            return c2

        lax.fori_loop(1, CPB // 2 - 1, pair_body, 0, unroll=False)
        chunk_work(CPB - 2, 0, True, True)
        chunk_work(CPB - 1, 1, False, True)
        return carry

    def zero_one(e, c2):
        for jj in range(H // 16):
            kd[1, e, pl.ds(jj * 16, 16)] = jnp.zeros((16,), jnp.float32)
        return c2

    lax.fori_loop(0, C, zero_one, 0, unroll=False)
    pltpu.sync_copy(src2_hbm.at[pl.ds(wid * EPW, EPB)], sidx)
    pltpu.sync_copy(dst2_hbm.at[pl.ds(wid * EPW, EPB)], didx)
    start_scatter(CPB - 1, 1)

    lax.fori_loop(0, NBLK, block_body, 0, unroll=False)
    wait_scatter(1)
    plsc.subcore_barrier()

    zbuf = kd.at[0, pl.ds(0, RCH)]
    for c in range(NRCH):
        r = row0 + c * RCH
        pltpu.sync_copy(aggsh.at[pl.ds(r, RCH)], zbuf)
        pltpu.sync_copy(zbuf, out_hbm.at[cid, pl.ds(r, RCH)])


@jax.jit
def _edge_pass(k, qv, src2, dst2, zeros):
    mesh = plsc.VectorSubcoreMesh(core_axis_name="c", subcore_axis_name="s")
    f = pl.kernel(
        _edge_body,
        out_type=jax.ShapeDtypeStruct((NC, NP, H), jnp.float32),
        mesh=mesh,
        scratch_types=[
            pltpu.VMEM((EPB,), jnp.int32),
            pltpu.VMEM((EPB,), jnp.int32),
            pltpu.VMEM((2, C, H), jnp.float32),
            pltpu.VMEM((2, C, 2 * H), jnp.float32),
            [pltpu.SemaphoreType.DMA, pltpu.SemaphoreType.DMA],
            [pltpu.SemaphoreType.DMA, pltpu.SemaphoreType.DMA],
            pltpu.VMEM_SHARED((NP, H), jnp.float32),
        ],
    )
    return f(k, qv, src2, dst2, zeros)


BLK = 2000


def _dense0_body(x_ref, wp_ref, bp_ref, wc_ref, bc_ref,
                 k_ref, qv_ref, s_ref):
    h = jnp.maximum(jnp.dot(x_ref[...], wp_ref[...],
                            preferred_element_type=jnp.float32)
                    + bp_ref[...], 0.0)
    out = jnp.dot(h, wc_ref[...],
                  preferred_element_type=jnp.float32) + bc_ref[...]
    k_ref[...] = out[:, 0:H]
    qv_ref[...] = out[:, H:3 * H]
    s_ref[...] = out[:, 3 * H:4 * H]


@jax.jit
def _dense0(x, wp, bp, wc, bc):
    return pl.pallas_call(
        _dense0_body,
        grid=(N // BLK,),
        in_specs=[
            pl.BlockSpec((BLK, H), lambda i: (i, 0)),
            pl.BlockSpec((H, H), lambda i: (0, 0)),
            pl.BlockSpec((1, H), lambda i: (0, 0)),
            pl.BlockSpec((H, 4 * H), lambda i: (0, 0)),
            pl.BlockSpec((1, 4 * H), lambda i: (0, 0)),
        ],
        out_specs=[
            pl.BlockSpec((BLK, H), lambda i: (i, 0)),
            pl.BlockSpec((BLK, 2 * H), lambda i: (i, 0)),
            pl.BlockSpec((BLK, H), lambda i: (i, 0)),
        ],
        out_shape=[
            jax.ShapeDtypeStruct((N, H), jnp.float32),
            jax.ShapeDtypeStruct((N, 2 * H), jnp.float32),
            jax.ShapeDtypeStruct((N, H), jnp.float32),
        ],
    )(x, wp, bp, wc, bc)


def _stats_body(a0_ref, a1_ref, s_ref, pre_ref, sum_ref, sq_ref):
    i = pl.program_id(0)
    pre = a0_ref[...] + a1_ref[...] + s_ref[...]
    pre_ref[...] = pre
    bs = jnp.sum(pre, axis=0, keepdims=True)
    bq = jnp.sum(pre * pre, axis=0, keepdims=True)

    @pl.when(i == 0)
    def _():
        sum_ref[...] = bs
        sq_ref[...] = bq

    @pl.when(i > 0)
    def _():
        sum_ref[...] += bs
        sq_ref[...] += bq


@jax.jit
def _stats(a0, a1, s):
    return pl.pallas_call(
        _stats_body,
        grid=(N // BLK,),
        in_specs=[pl.BlockSpec((BLK, H), lambda i: (i, 0))] * 3,
        out_specs=[
            pl.BlockSpec((BLK, H), lambda i: (i, 0)),
            pl.BlockSpec((1, H), lambda i: (0, 0)),
            pl.BlockSpec((1, H), lambda i: (0, 0)),
        ],
        out_shape=[
            jax.ShapeDtypeStruct((N, H), jnp.float32),
            jax.ShapeDtypeStruct((1, H), jnp.float32),
            jax.ShapeDtypeStruct((1, H), jnp.float32),
        ],
    )(a0, a1, s)


def _normproj_body(pre_ref, sum_ref, sq_ref, g_ref, be_ref, wc_ref, bc_ref,
                   k_ref, qv_ref, s_ref):
    mu = sum_ref[...] / N
    var = sq_ref[...] / N - mu * mu
    scale = g_ref[...] * lax.rsqrt(var + 1e-5)
    h = jnp.maximum((pre_ref[...] - mu) * scale + be_ref[...], 0.0)
    out = jnp.dot(h, wc_ref[...],
                  preferred_element_type=jnp.float32) + bc_ref[...]
    k_ref[...] = out[:, 0:H]
    qv_ref[...] = out[:, H:3 * H]
    s_ref[...] = out[:, 3 * H:4 * H]


@jax.jit
def _normproj(pre, sm, sq, g, be, wc, bc):
    return pl.pallas_call(
        _normproj_body,
        grid=(N // BLK,),
        in_specs=[
            pl.BlockSpec((BLK, H), lambda i: (i, 0)),
            pl.BlockSpec((1, H), lambda i: (0, 0)),
            pl.BlockSpec((1, H), lambda i: (0, 0)),
            pl.BlockSpec((1, H), lambda i: (0, 0)),
            pl.BlockSpec((1, H), lambda i: (0, 0)),
            pl.BlockSpec((H, 4 * H), lambda i: (0, 0)),
            pl.BlockSpec((1, 4 * H), lambda i: (0, 0)),
        ],
        out_specs=[
            pl.BlockSpec((BLK, H), lambda i: (i, 0)),
            pl.BlockSpec((BLK, 2 * H), lambda i: (i, 0)),
            pl.BlockSpec((BLK, H), lambda i: (i, 0)),
        ],
        out_shape=[
            jax.ShapeDtypeStruct((N, H), jnp.float32),
            jax.ShapeDtypeStruct((N, 2 * H), jnp.float32),
            jax.ShapeDtypeStruct((N, H), jnp.float32),
        ],
    )(pre, sm, sq, g, be, wc, bc)


def _head_body(pre_ref, sum_ref, sq_ref, g_ref, be_ref, wc_ref, bc_ref,
               out_ref):
    mu = sum_ref[...] / N
    var = sq_ref[...] / N - mu * mu
    scale = g_ref[...] * lax.rsqrt(var + 1e-5)
    h = jnp.maximum((pre_ref[...] - mu) * scale + be_ref[...], 0.0)
    out_ref[...] = jnp.dot(h, wc_ref[...],
                           preferred_element_type=jnp.float32) + bc_ref[...]


@jax.jit
def _head(pre, sm, sq, g, be, wc, bc):
    m = wc.shape[1]
    return pl.pallas_call(
        _head_body,
        grid=(N // BLK,),
        in_specs=[
            pl.BlockSpec((BLK, H), lambda i: (i, 0)),
            pl.BlockSpec((1, H), lambda i: (0, 0)),
            pl.BlockSpec((1, H), lambda i: (0, 0)),
            pl.BlockSpec((1, H), lambda i: (0, 0)),
            pl.BlockSpec((1, H), lambda i: (0, 0)),
            pl.BlockSpec((H, m), lambda i: (0, 0)),
            pl.BlockSpec((1, m), lambda i: (0, 0)),
        ],
        out_specs=pl.BlockSpec((BLK, m), lambda i: (i, 0)),
        out_shape=jax.ShapeDtypeStruct((N, m), jnp.float32),
    )(pre, sm, sq, g, be, wc, bc)


def _wcat(c):
    wc = jnp.concatenate([c['Wk'], c['Wq'], c['Wv'], c['Ws']], axis=1)
    bc = jnp.concatenate([c['bk'], c['bq'], c['bv'], c['b']])[None, :]
    return wc, bc


def kernel(x, ei, params):
    p = params
    zeros = jnp.zeros((RCH, H), jnp.float32)
    src2 = ei[0]
    dst2 = ei[1]

    wc1, bc1 = _wcat(p['c1'])
    k, qv, s = _dense0(x, p['Wp'], p['bp'][None, :], wc1, bc1)

    for i in (1, 2, 3):
        aggp = _edge_pass(k, qv, src2, dst2, zeros)
        pre, sm, sq = _stats(aggp[0, :N], aggp[1, :N], s)
        g = p['g%d' % i][None, :]
        be = p['be%d' % i][None, :]
        if i < 3:
            wc, bc = _wcat(p['c%d' % (i + 1)])
            k, qv, s = _normproj(pre, sm, sq, g, be, wc, bc)
        else:
            out = _head(pre, sm, sq, g, be, p['Wh'], p['bh'][None, :])
    return out


# speedup vs baseline: 3.9503x; 3.3455x over previous
import jax
import jax.numpy as jnp
from jax import lax
from jax.experimental import pallas as pl
from jax.experimental.pallas import tpu as pltpu
from jax.experimental.pallas import tpu_sc as plsc

N = 10000
E = 320000
H = 128

NC = 2
NS = 16
NW = NC * NS
EPW = E // NW
C = 40
CPW = EPW // C
CPB = 50
EPB = C * CPB
NBLK = CPW // CPB
NP = 10240
RPT = NP // NS
RCH = 40
NRCH = RPT // RCH


def _edge_body(k_hbm, qv_hbm, src2_hbm, dst2_hbm, zeros_hbm, out_hbm,
               sidx, didx, kd, qvd, msgb, gsems, ssems, aggsh):
    cid = lax.axis_index("c")
    sid = lax.axis_index("s")
    wid = sid * NC + cid

    row0 = sid * RPT
    for c in range(NRCH):
        pltpu.sync_copy(zeros_hbm, aggsh.at[pl.ds(row0 + c * RCH, RCH)])
    plsc.subcore_barrier()

    crow0 = wid * CPW

    def start_gathers(j, b):
        sl = pl.ds(j * C, C)
        pltpu.async_copy(k_hbm.at[didx.at[sl]], kd.at[b], gsems[b])
        pltpu.async_copy(qv_hbm.at[sidx.at[sl]], qvd.at[b], gsems[b])

    def wait_gathers(j, b):
        sl = pl.ds(j * C, C)
        pltpu.make_async_copy(k_hbm.at[didx.at[sl]], kd.at[b],
                              gsems[b]).wait()
        pltpu.make_async_copy(qv_hbm.at[sidx.at[sl]], qvd.at[b],
                              gsems[b]).wait()

    def start_scatter(j, b):
        pltpu.async_copy(msgb.at[b], aggsh.at[didx.at[pl.ds(j * C, C)]],
                         ssems[b], add=True)

    def wait_scatter(b):
        pltpu.make_async_copy(msgb.at[b], aggsh.at[didx.at[pl.ds(0, C)]],
                              ssems[b]).wait()

    def compute_chunk(b):
        kb = kd.at[b]
        qb = qvd.at[b]
        mb = msgb.at[b]

        @plsc.parallel_loop(0, C, 1, unroll=4)
        def edge_one(e):
            for jj in range(H // 16):
                sl = pl.ds(jj * 16, 16)
                kk = kb[e, sl]
                qq = qb[e, sl]
                vv = qb[e, pl.ds(H + jj * 16, 16)]
                em = jnp.exp(-(kk + qq))
                mb[e, sl] = vv / (1.0 + em)

    def chunk_work(j, b, prefetch, drain):
        wait_gathers(j, b)
        if drain:
            wait_scatter(1 - b)
        if prefetch:
            start_gathers(j + 1, 1 - b)
        compute_chunk(b)
        start_scatter(j, b)

    def block_body(nb, carry):
        wait_scatter(1)
        base = wid * EPW + nb * EPB
        pltpu.sync_copy(src2_hbm.at[pl.ds(base, EPB)], sidx)
        pltpu.sync_copy(dst2_hbm.at[pl.ds(base, EPB)], didx)
        start_gathers(0, 0)
        chunk_work(0, 0, True, False)
        chunk_work(1, 1, True, True)

        def pair_body(t, c2):
            chunk_work(2 * t, 0, True, True)
            chunk_work(2 * t + 1, 1, True, True)
            return c2

        lax.fori_loop(1, CPB // 2 - 1, pair_body, 0, unroll=False)
        chunk_work(CPB - 2, 0, True, True)
        chunk_work(CPB - 1, 1, False, True)
        return carry

    def zero_one(e, c2):
        for jj in range(H // 16):
            msgb[1, e, pl.ds(jj * 16, 16)] = jnp.zeros((16,), jnp.float32)
        return c2

    lax.fori_loop(0, C, zero_one, 0, unroll=False)
    pltpu.sync_copy(src2_hbm.at[pl.ds(wid * EPW, EPB)], sidx)
    pltpu.sync_copy(dst2_hbm.at[pl.ds(wid * EPW, EPB)], didx)
    start_scatter(CPB - 1, 1)

    lax.fori_loop(0, NBLK, block_body, 0, unroll=False)
    wait_scatter(1)
    plsc.subcore_barrier()

    zbuf = kd.at[0, pl.ds(0, RCH)]
    for c in range(NRCH):
        r = row0 + c * RCH
        pltpu.sync_copy(aggsh.at[pl.ds(r, RCH)], zbuf)
        pltpu.sync_copy(zbuf, out_hbm.at[cid, pl.ds(r, RCH)])


@jax.jit
def _edge_pass(k, qv, src2, dst2, zeros):
    mesh = plsc.VectorSubcoreMesh(core_axis_name="c", subcore_axis_name="s")
    f = pl.kernel(
        _edge_body,
        out_type=jax.ShapeDtypeStruct((NC, NP, H), jnp.float32),
        mesh=mesh,
        scratch_types=[
            pltpu.VMEM((EPB,), jnp.int32),
            pltpu.VMEM((EPB,), jnp.int32),
            pltpu.VMEM((2, C, H), jnp.float32),
            pltpu.VMEM((2, C, 2 * H), jnp.float32),
            pltpu.VMEM((2, C, H), jnp.float32),
            [pltpu.SemaphoreType.DMA, pltpu.SemaphoreType.DMA],
            [pltpu.SemaphoreType.DMA, pltpu.SemaphoreType.DMA],
            pltpu.VMEM_SHARED((NP, H), jnp.float32),
        ],
    )
    return f(k, qv, src2, dst2, zeros)


BLK = 2000


def _dense0_body(x_ref, wp_ref, bp_ref, wc_ref, bc_ref,
                 k_ref, qv_ref, s_ref):
    h = jnp.maximum(jnp.dot(x_ref[...], wp_ref[...],
                            preferred_element_type=jnp.float32)
                    + bp_ref[...], 0.0)
    out = jnp.dot(h, wc_ref[...],
                  preferred_element_type=jnp.float32) + bc_ref[...]
    k_ref[...] = out[:, 0:H]
    qv_ref[...] = out[:, H:3 * H]
    s_ref[...] = out[:, 3 * H:4 * H]


@jax.jit
def _dense0(x, wp, bp, wc, bc):
    return pl.pallas_call(
        _dense0_body,
        grid=(N // BLK,),
        in_specs=[
            pl.BlockSpec((BLK, H), lambda i: (i, 0)),
            pl.BlockSpec((H, H), lambda i: (0, 0)),
            pl.BlockSpec((1, H), lambda i: (0, 0)),
            pl.BlockSpec((H, 4 * H), lambda i: (0, 0)),
            pl.BlockSpec((1, 4 * H), lambda i: (0, 0)),
        ],
        out_specs=[
            pl.BlockSpec((BLK, H), lambda i: (i, 0)),
            pl.BlockSpec((BLK, 2 * H), lambda i: (i, 0)),
            pl.BlockSpec((BLK, H), lambda i: (i, 0)),
        ],
        out_shape=[
            jax.ShapeDtypeStruct((N, H), jnp.float32),
            jax.ShapeDtypeStruct((N, 2 * H), jnp.float32),
            jax.ShapeDtypeStruct((N, H), jnp.float32),
        ],
    )(x, wp, bp, wc, bc)


def _stats_body(a0_ref, a1_ref, s_ref, pre_ref, sum_ref, sq_ref):
    i = pl.program_id(0)
    pre = a0_ref[...] + a1_ref[...] + s_ref[...]
    pre_ref[...] = pre
    bs = jnp.sum(pre, axis=0, keepdims=True)
    bq = jnp.sum(pre * pre, axis=0, keepdims=True)

    @pl.when(i == 0)
    def _():
        sum_ref[...] = bs
        sq_ref[...] = bq

    @pl.when(i > 0)
    def _():
        sum_ref[...] += bs
        sq_ref[...] += bq


@jax.jit
def _stats(a0, a1, s):
    return pl.pallas_call(
        _stats_body,
        grid=(N // BLK,),
        in_specs=[pl.BlockSpec((BLK, H), lambda i: (i, 0))] * 3,
        out_specs=[
            pl.BlockSpec((BLK, H), lambda i: (i, 0)),
            pl.BlockSpec((1, H), lambda i: (0, 0)),
            pl.BlockSpec((1, H), lambda i: (0, 0)),
        ],
        out_shape=[
            jax.ShapeDtypeStruct((N, H), jnp.float32),
            jax.ShapeDtypeStruct((1, H), jnp.float32),
            jax.ShapeDtypeStruct((1, H), jnp.float32),
        ],
    )(a0, a1, s)


def _normproj_body(pre_ref, sum_ref, sq_ref, g_ref, be_ref, wc_ref, bc_ref,
                   k_ref, qv_ref, s_ref):
    mu = sum_ref[...] / N
    var = sq_ref[...] / N - mu * mu
    scale = g_ref[...] * lax.rsqrt(var + 1e-5)
    h = jnp.maximum((pre_ref[...] - mu) * scale + be_ref[...], 0.0)
    out = jnp.dot(h, wc_ref[...],
                  preferred_element_type=jnp.float32) + bc_ref[...]
    k_ref[...] = out[:, 0:H]
    qv_ref[...] = out[:, H:3 * H]
    s_ref[...] = out[:, 3 * H:4 * H]


@jax.jit
def _normproj(pre, sm, sq, g, be, wc, bc):
    return pl.pallas_call(
        _normproj_body,
        grid=(N // BLK,),
        in_specs=[
            pl.BlockSpec((BLK, H), lambda i: (i, 0)),
            pl.BlockSpec((1, H), lambda i: (0, 0)),
            pl.BlockSpec((1, H), lambda i: (0, 0)),
            pl.BlockSpec((1, H), lambda i: (0, 0)),
            pl.BlockSpec((1, H), lambda i: (0, 0)),
            pl.BlockSpec((H, 4 * H), lambda i: (0, 0)),
            pl.BlockSpec((1, 4 * H), lambda i: (0, 0)),
        ],
        out_specs=[
            pl.BlockSpec((BLK, H), lambda i: (i, 0)),
            pl.BlockSpec((BLK, 2 * H), lambda i: (i, 0)),
            pl.BlockSpec((BLK, H), lambda i: (i, 0)),
        ],
        out_shape=[
            jax.ShapeDtypeStruct((N, H), jnp.float32),
            jax.ShapeDtypeStruct((N, 2 * H), jnp.float32),
            jax.ShapeDtypeStruct((N, H), jnp.float32),
        ],
    )(pre, sm, sq, g, be, wc, bc)


def _head_body(pre_ref, sum_ref, sq_ref, g_ref, be_ref, wc_ref, bc_ref,
               out_ref):
    mu = sum_ref[...] / N
    var = sq_ref[...] / N - mu * mu
    scale = g_ref[...] * lax.rsqrt(var + 1e-5)
    h = jnp.maximum((pre_ref[...] - mu) * scale + be_ref[...], 0.0)
    out_ref[...] = jnp.dot(h, wc_ref[...],
                           preferred_element_type=jnp.float32) + bc_ref[...]


@jax.jit
def _head(pre, sm, sq, g, be, wc, bc):
    m = wc.shape[1]
    return pl.pallas_call(
        _head_body,
        grid=(N // BLK,),
        in_specs=[
            pl.BlockSpec((BLK, H), lambda i: (i, 0)),
            pl.BlockSpec((1, H), lambda i: (0, 0)),
            pl.BlockSpec((1, H), lambda i: (0, 0)),
            pl.BlockSpec((1, H), lambda i: (0, 0)),
            pl.BlockSpec((1, H), lambda i: (0, 0)),
            pl.BlockSpec((H, m), lambda i: (0, 0)),
            pl.BlockSpec((1, m), lambda i: (0, 0)),
        ],
        out_specs=pl.BlockSpec((BLK, m), lambda i: (i, 0)),
        out_shape=jax.ShapeDtypeStruct((N, m), jnp.float32),
    )(pre, sm, sq, g, be, wc, bc)


def _wcat(c):
    wc = jnp.concatenate([c['Wk'], c['Wq'], c['Wv'], c['Ws']], axis=1)
    bc = jnp.concatenate([c['bk'], c['bq'], c['bv'], c['b']])[None, :]
    return wc, bc


def kernel(x, ei, params):
    p = params
    zeros = jnp.zeros((RCH, H), jnp.float32)
    src2 = ei[0]
    dst2 = ei[1]

    wc1, bc1 = _wcat(p['c1'])
    k, qv, s = _dense0(x, p['Wp'], p['bp'][None, :], wc1, bc1)

    for i in (1, 2, 3):
        aggp = _edge_pass(k, qv, src2, dst2, zeros)
        pre, sm, sq = _stats(aggp[0, :N], aggp[1, :N], s)
        g = p['g%d' % i][None, :]
        be = p['be%d' % i][None, :]
        if i < 3:
            wc, bc = _wcat(p['c%d' % (i + 1)])
            k, qv, s = _normproj(pre, sm, sq, g, be, wc, bc)
        else:
            out = _head(pre, sm, sq, g, be, p['Wh'], p['bh'][None, :])
    return out
